# baseline (speedup 1.0000x reference)
import functools

import jax
import jax.numpy as jnp
from jax import lax
from jax.experimental import pallas as pl
from jax.experimental.pallas import tpu as pltpu
from jax.experimental.pallas import tpu_sc as plsc

_K = 1024
_TOK_BLOCK = 2048


def _dist_kernel(z_ref, e_ref, idx_ref, sse_ref):
    z = z_ref[...]
    e = e_ref[...]
    a = jnp.sum(z * z, axis=1, keepdims=True)
    em2 = e * -2.0
    m2 = jax.lax.dot_general(z, em2, (((1,), (1,)), ((), ())))
    b = jnp.sum(e * e, axis=1, keepdims=True).T
    dists = a + m2 + b
    mins = jnp.min(dists, axis=1, keepdims=True)
    ks = jax.lax.broadcasted_iota(jnp.int32, (1, _K), 1)
    idx = jnp.min(jnp.where(dists == mins, ks, _K), axis=1)
    idx_ref[...] = idx
    sse_ref[...] = jnp.sum(mins).reshape(1, 1, 1)


def _make_sc_gather(n_tok, d):
    info = plsc.get_sparse_core_info()
    nw = info.num_cores * info.num_subcores
    b_per_w = n_tok // nw
    mesh = plsc.VectorSubcoreMesh(core_axis_name="c", subcore_axis_name="s")

    @functools.partial(
        pl.kernel, mesh=mesh,
        out_type=jax.ShapeDtypeStruct((n_tok, d), jnp.float32),
        compiler_params=pltpu.CompilerParams(use_tc_tiling_on_sc=False),
        scratch_types=[
            pltpu.VMEM((b_per_w,), jnp.int32),
            pltpu.VMEM((b_per_w, d), jnp.float32),
            pltpu.SemaphoreType.DMA,
        ],
    )
    def _gather(table_hbm, idx_hbm, out_hbm, idx_v, rows_v, sem):
        wid = lax.axis_index("s") * info.num_cores + lax.axis_index("c")
        base = wid * b_per_w
        pltpu.sync_copy(idx_hbm.at[pl.ds(base, b_per_w)], idx_v)
        pltpu.async_copy(table_hbm.at[idx_v], rows_v, sem).wait()
        pltpu.sync_copy(rows_v, out_hbm.at[pl.ds(base, b_per_w)])

    return _gather


def kernel(z_e, embedding_weight):
    B, D, H, W = z_e.shape
    N = B * H * W
    z_flat = jnp.transpose(z_e, (0, 2, 3, 1)).reshape(N, D)
    nblk = N // _TOK_BLOCK
    idx, sse = pl.pallas_call(
        _dist_kernel,
        grid=(nblk,),
        in_specs=[
            pl.BlockSpec((_TOK_BLOCK, D), lambda i: (i, 0)),
            pl.BlockSpec((_K, D), lambda i: (0, 0)),
        ],
        out_specs=[
            pl.BlockSpec((_TOK_BLOCK,), lambda i: (i,)),
            pl.BlockSpec((1, 1, 1), lambda i: (i, 0, 0)),
        ],
        out_shape=[
            jax.ShapeDtypeStruct((N,), jnp.int32),
            jax.ShapeDtypeStruct((nblk, 1, 1), jnp.float32),
        ],
        compiler_params=pltpu.CompilerParams(
            allow_input_fusion=[True, False],
            dimension_semantics=["parallel"]),
    )(z_flat, embedding_weight)
    zq_flat = _make_sc_gather(N, D)(embedding_weight, idx)
    inv = 1.0 / (N * D)
    stot = jnp.sum(sse)
    codebook_loss = (stot * inv).astype(jnp.float32)
    commitment_loss = (stot * (0.25 * inv)).astype(jnp.float32)
    z_q = jnp.transpose(zq_flat.reshape(B, H, W, D), (0, 3, 1, 2))
    return z_q, codebook_loss, commitment_loss


# speedup vs baseline: 1.0142x; 1.0142x over previous
import functools

import jax
import jax.numpy as jnp
from jax import lax
from jax.experimental import pallas as pl
from jax.experimental.pallas import tpu as pltpu
from jax.experimental.pallas import tpu_sc as plsc

_K = 1024
_TOK_BLOCK = 4096


def _dist_kernel(z_ref, e_ref, idx_ref, sse_ref):
    z = z_ref[...]
    e = e_ref[...]
    a = jnp.sum(z * z, axis=1, keepdims=True)
    em2 = e * -2.0
    m2 = jax.lax.dot_general(z, em2, (((1,), (1,)), ((), ())))
    b = jnp.sum(e * e, axis=1, keepdims=True).T
    dists = a + m2 + b
    mins = jnp.min(dists, axis=1, keepdims=True)
    ks = jax.lax.broadcasted_iota(jnp.int32, (1, _K), 1)
    idx = jnp.min(jnp.where(dists == mins, ks, _K), axis=1)
    idx_ref[...] = idx
    sse_ref[...] = jnp.sum(mins).reshape(1, 1, 1)


def _make_sc_gather(n_tok, d):
    info = plsc.get_sparse_core_info()
    nw = info.num_cores * info.num_subcores
    b_per_w = n_tok // nw
    mesh = plsc.VectorSubcoreMesh(core_axis_name="c", subcore_axis_name="s")

    @functools.partial(
        pl.kernel, mesh=mesh,
        out_type=jax.ShapeDtypeStruct((n_tok, d), jnp.float32),
        compiler_params=pltpu.CompilerParams(use_tc_tiling_on_sc=False),
        scratch_types=[
            pltpu.VMEM((b_per_w,), jnp.int32),
            pltpu.VMEM((b_per_w, d), jnp.float32),
            pltpu.SemaphoreType.DMA,
        ],
    )
    def _gather(table_hbm, idx_hbm, out_hbm, idx_v, rows_v, sem):
        wid = lax.axis_index("s") * info.num_cores + lax.axis_index("c")
        base = wid * b_per_w
        pltpu.sync_copy(idx_hbm.at[pl.ds(base, b_per_w)], idx_v)
        pltpu.async_copy(table_hbm.at[idx_v], rows_v, sem).wait()
        pltpu.sync_copy(rows_v, out_hbm.at[pl.ds(base, b_per_w)])

    return _gather


def kernel(z_e, embedding_weight):
    B, D, H, W = z_e.shape
    N = B * H * W
    z_flat = jnp.transpose(z_e, (0, 2, 3, 1)).reshape(N, D)
    nblk = N // _TOK_BLOCK
    idx, sse = pl.pallas_call(
        _dist_kernel,
        grid=(nblk,),
        in_specs=[
            pl.BlockSpec((_TOK_BLOCK, D), lambda i: (i, 0)),
            pl.BlockSpec((_K, D), lambda i: (0, 0)),
        ],
        out_specs=[
            pl.BlockSpec((_TOK_BLOCK,), lambda i: (i,)),
            pl.BlockSpec((1, 1, 1), lambda i: (i, 0, 0)),
        ],
        out_shape=[
            jax.ShapeDtypeStruct((N,), jnp.int32),
            jax.ShapeDtypeStruct((nblk, 1, 1), jnp.float32),
        ],
        compiler_params=pltpu.CompilerParams(
            allow_input_fusion=[True, False],
            dimension_semantics=["parallel"]),
    )(z_flat, embedding_weight)
    zq_flat = _make_sc_gather(N, D)(embedding_weight, idx)
    inv = 1.0 / (N * D)
    stot = jnp.sum(sse)
    codebook_loss = (stot * inv).astype(jnp.float32)
    commitment_loss = (stot * (0.25 * inv)).astype(jnp.float32)
    z_q = jnp.transpose(zq_flat.reshape(B, H, W, D), (0, 3, 1, 2))
    return z_q, codebook_loss, commitment_loss


# speedup vs baseline: 1.0168x; 1.0026x over previous
import functools

import jax
import jax.numpy as jnp
from jax import lax
from jax.experimental import pallas as pl
from jax.experimental.pallas import tpu as pltpu
from jax.experimental.pallas import tpu_sc as plsc

_K = 1024
_TOK_BLOCK = 4096


def _dist_kernel(z_ref, e_ref, idx_ref, sse_ref):
    z = z_ref[...]
    e = e_ref[...]
    a = jnp.sum(z * z, axis=1, keepdims=True)
    em2 = e * -2.0
    m2 = jax.lax.dot_general(z, em2, (((1,), (1,)), ((), ())))
    b = jnp.sum(e * e, axis=1, keepdims=True).T
    dists = a + m2 + b
    mins = jnp.min(dists, axis=1, keepdims=True)
    ks = jax.lax.broadcasted_iota(jnp.int32, (1, _K), 1)
    idx = jnp.min(jnp.where(dists == mins, ks, _K), axis=1)
    idx_ref[...] = idx
    sse_ref[...] = jnp.sum(mins).reshape(1, 1, 1)


def _make_sc_gather(n_tok, d):
    info = plsc.get_sparse_core_info()
    nw = info.num_cores * info.num_subcores
    b_per_w = n_tok // nw
    mesh = plsc.VectorSubcoreMesh(core_axis_name="c", subcore_axis_name="s")

    @functools.partial(
        pl.kernel, mesh=mesh,
        out_type=jax.ShapeDtypeStruct((n_tok, d), jnp.float32),
        compiler_params=pltpu.CompilerParams(use_tc_tiling_on_sc=False),
        scratch_types=[
            pltpu.VMEM((b_per_w,), jnp.int32),
            pltpu.VMEM((b_per_w, d), jnp.float32),
            pltpu.SemaphoreType.DMA,
        ],
    )
    def _gather(table_hbm, idx_hbm, out_hbm, idx_v, rows_v, sem):
        wid = lax.axis_index("s") * info.num_cores + lax.axis_index("c")
        base = wid * b_per_w
        pltpu.sync_copy(idx_hbm.at[pl.ds(base, b_per_w)], idx_v)
        pltpu.async_copy(table_hbm.at[idx_v], rows_v, sem).wait()
        pltpu.sync_copy(rows_v, out_hbm.at[pl.ds(base, b_per_w)])

    return _gather


def kernel(z_e, embedding_weight):
    B, D, H, W = z_e.shape
    N = B * H * W
    z_flat = jnp.transpose(z_e, (0, 2, 3, 1)).reshape(N, D)
    nblk = N // _TOK_BLOCK
    idx, sse = pl.pallas_call(
        _dist_kernel,
        grid=(nblk,),
        in_specs=[
            pl.BlockSpec((_TOK_BLOCK, D), lambda i: (i, 0)),
            pl.BlockSpec((_K, D), lambda i: (0, 0)),
        ],
        out_specs=[
            pl.BlockSpec((_TOK_BLOCK,), lambda i: (i,)),
            pl.BlockSpec((1, 1, 1), lambda i: (i, 0, 0)),
        ],
        out_shape=[
            jax.ShapeDtypeStruct((N,), jnp.int32),
            jax.ShapeDtypeStruct((nblk, 1, 1), jnp.float32),
        ],
        compiler_params=pltpu.CompilerParams(
            dimension_semantics=["parallel"]),
    )(z_flat, embedding_weight)
    zq_flat = _make_sc_gather(N, D)(embedding_weight, idx)
    inv = 1.0 / (N * D)
    stot = jnp.sum(sse)
    codebook_loss = (stot * inv).astype(jnp.float32)
    commitment_loss = (stot * (0.25 * inv)).astype(jnp.float32)
    z_q = jnp.transpose(zq_flat.reshape(B, H, W, D), (0, 3, 1, 2))
    return z_q, codebook_loss, commitment_loss
